# baseline (speedup 1.0000x reference)
import jax
import jax.numpy as jnp
from jax import lax
from jax.experimental import pallas as pl
from jax.experimental.pallas import tpu as pltpu
from jax.experimental.pallas import tpu_sc as plsc

B, C, H, W = 16, 20, 128, 128
N, DIN, D = 512, 256, 32
HW = H * W
NQ = B // 4
QROWS_PAD = HW + 64
SC_ROWS = NQ * QROWS_PAD * 4
QBASE = QROWS_PAD * 4
SC_ELEMS = SC_ROWS * 32
NWORK = 32
ZELEMS = SC_ELEMS // NWORK
ZBUF = 32768
HWT = 4096


def _ka_body(emb_ref, fidx_ref, w_ref, b_ref, projt_ref, ridx_ref):
    b = pl.program_id(0)
    q = b // 4
    j = b % 4
    emb = emb_ref[0]
    proj = lax.dot_general(
        emb, w_ref[...], (((1,), (0,)), ((), ())),
        preferred_element_type=jnp.float32,
        precision=lax.Precision.HIGHEST,
    ) + b_ref[...]
    t = jnp.swapaxes(jnp.maximum(proj, 0.0), 0, 1)
    projt_ref[0, 0] = t[:, :N // 2]
    projt_ref[0, 1] = t[:, N // 2:]

    row = fidx_ref[0]
    col = jnp.swapaxes(row, 0, 1)
    eq = col == row
    later = (lax.broadcasted_iota(jnp.int32, (N, N), 1)
             > lax.broadcasted_iota(jnp.int32, (N, N), 0))
    dup = jnp.any(eq & later, axis=1, keepdims=True)
    dest = q * QBASE + 4 * col + j
    dump = q * QBASE + 4 * HW + j
    ridx_ref[0] = jnp.where(dup, dump, dest)


def _ksc_body(projt_hbm, ridx_hbm, scat_hbm, zbuf, idx2, idxb, val_v, sem):
    c = lax.axis_index("c")
    s = lax.axis_index("s")
    w = c * 16 + s

    z = jnp.zeros((16,), jnp.float32)

    def zstep(i, carry):
        for r in range(8):
            zbuf[pl.ds((i * 8 + r) * 16, 16)] = z
        return carry

    lax.fori_loop(0, ZBUF // 128, zstep, 0)
    base = w * ZELEMS
    cps = [pltpu.async_copy(zbuf, scat_hbm.at[pl.ds(base + k * ZBUF, ZBUF)], sem)
           for k in range(8)]
    rem = ZELEMS - 8 * ZBUF
    cps.append(pltpu.async_copy(zbuf.at[pl.ds(0, rem)],
                                scat_hbm.at[pl.ds(base + 8 * ZBUF, rem)], sem))
    for cp in cps:
        cp.wait()

    plsc.subcore_barrier()

    b = c * 8 + s // 2
    half = s % 2
    pltpu.sync_copy(ridx_hbm.at[b, pl.ds(half * 2, 2)], idx2)
    pltpu.sync_copy(projt_hbm.at[b, half], val_v)

    for hh in range(2):
        for g in range(8):
            v32 = idx2[hh, pl.ds(g * 16, 16)] * 32
            for d in range(D):
                idxb[d * 2 + hh, pl.ds(g * 16, 16)] = v32 + d

    for k0 in range(0, 2 * D, 16):
        cps = []
        for k in range(k0, k0 + 16):
            d, hh = k // 2, k % 2
            cps.append(pltpu.async_copy(val_v.at[d, pl.ds(hh * 128, 128)],
                                        scat_hbm.at[idxb.at[k]], sem))
        for cp in cps:
            cp.wait()


def _kc_body(scat_ref, sp_ref, out_ref):
    t = jnp.swapaxes(scat_ref[0], 0, 1)
    for j in range(4):
        out_ref[j, :C, :] = sp_ref[j]
        out_ref[j, C:, :] = t[j * 32:(j + 1) * 32, :]


def kernel(spatial_info, entity_embeddings, locations, W_proj, b_proj):
    lh = jnp.clip(locations[..., 0], 0, H - 1)
    lw = jnp.clip(locations[..., 1], 0, W - 1)
    fidx = (lh * W + lw).astype(jnp.int32).reshape(B, 1, N)
    sp = spatial_info.reshape(B, C, HW)
    b2 = b_proj.reshape(1, D)

    projt, ridx = pl.pallas_call(
        _ka_body,
        grid=(B,),
        in_specs=[
            pl.BlockSpec((1, N, DIN), lambda b: (b, 0, 0)),
            pl.BlockSpec((1, 1, N), lambda b: (b, 0, 0)),
            pl.BlockSpec((DIN, D), lambda b: (0, 0)),
            pl.BlockSpec((1, D), lambda b: (0, 0)),
        ],
        out_specs=[
            pl.BlockSpec((1, 2, D, N // 2), lambda b: (b, 0, 0, 0)),
            pl.BlockSpec((1, N, 1), lambda b: (b, 0, 0)),
        ],
        out_shape=[
            jax.ShapeDtypeStruct((B, 2, D, N // 2), jnp.float32),
            jax.ShapeDtypeStruct((B, N, 1), jnp.int32),
        ],
    )(entity_embeddings, fidx, W_proj, b2)
    ridx = ridx.reshape(B, 4, 128)

    mesh = plsc.VectorSubcoreMesh(core_axis_name="c", subcore_axis_name="s")
    scat = pl.kernel(
        _ksc_body,
        out_type=jax.ShapeDtypeStruct((SC_ELEMS,), jnp.float32),
        mesh=mesh,
        scratch_types=[
            pltpu.VMEM((ZBUF,), jnp.float32),
            pltpu.VMEM((2, 128), jnp.int32),
            pltpu.VMEM((2 * D, 128), jnp.int32),
            pltpu.VMEM((D, 256), jnp.float32),
            pltpu.SemaphoreType.DMA,
        ],
    )(projt, ridx)
    scatq = scat.reshape(NQ, QROWS_PAD, 128)

    out_flat = pl.pallas_call(
        _kc_body,
        grid=(NQ, HW // HWT),
        in_specs=[
            pl.BlockSpec((1, HWT, 128), lambda q, h: (q, h, 0)),
            pl.BlockSpec((4, C, HWT), lambda q, h: (q, 0, h)),
        ],
        out_specs=pl.BlockSpec((4, C + D, HWT), lambda q, h: (q, 0, h)),
        out_shape=jax.ShapeDtypeStruct((B, C + D, HW), jnp.float32),
    )(scatq, sp)
    return out_flat.reshape(B, C + D, H, W)


# speedup vs baseline: 3.2541x; 3.2541x over previous
import jax
import jax.numpy as jnp
from jax import lax
from jax.experimental import pallas as pl
from jax.experimental.pallas import tpu as pltpu
from jax.experimental.pallas import tpu_sc as plsc

B, C, H, W = 16, 20, 128, 128
N, DIN, D = 512, 256, 32
HW = H * W
NQ = B // 4
QROWS_PAD = HW + 64
SC_ROWS = NQ * QROWS_PAD * 4
QBASE = QROWS_PAD * 4
NWORK = 32
ZLROWS = SC_ROWS // NWORK
ZBUF = 1024
HWT = 4096


def _ka_body(emb_ref, fidx_ref, w_ref, b_ref, proj_ref, ridx_ref):
    b = pl.program_id(0)
    q = b // 4
    j = b % 4
    emb = emb_ref[0]
    proj = lax.dot_general(
        emb, w_ref[...], (((1,), (0,)), ((), ())),
        preferred_element_type=jnp.float32,
        precision=lax.Precision.HIGHEST,
    ) + b_ref[...]
    proj_ref[0] = jnp.maximum(proj, 0.0)

    row = fidx_ref[0]
    col = jnp.swapaxes(row, 0, 1)
    eq = col == row
    later = (lax.broadcasted_iota(jnp.int32, (N, N), 1)
             > lax.broadcasted_iota(jnp.int32, (N, N), 0))
    dup = jnp.any(eq & later, axis=1, keepdims=True)
    dest = q * QBASE + 4 * col + j
    dump = q * QBASE + 4 * HW + j
    ridx_ref[0] = jnp.where(dup, dump, dest)


def _ksc_body(proj_hbm, ridx_hbm, scat_hbm, zbuf, idx2, val_v, sem):
    c = lax.axis_index("c")
    s = lax.axis_index("s")
    w = c * 16 + s

    z = jnp.zeros((16,), jnp.float32)

    def zstep(i, carry):
        for r in range(8):
            for g in range(2):
                zbuf[i * 8 + r, pl.ds(g * 16, 16)] = z
        return carry

    lax.fori_loop(0, ZBUF // 8, zstep, 0)
    base = w * ZLROWS
    cps = [pltpu.async_copy(zbuf, scat_hbm.at[pl.ds(base + k * ZBUF, ZBUF)], sem)
           for k in range(8)]
    rem = ZLROWS - 8 * ZBUF
    cps.append(pltpu.async_copy(zbuf.at[pl.ds(0, rem)],
                                scat_hbm.at[pl.ds(base + 8 * ZBUF, rem)], sem))
    for cp in cps:
        cp.wait()

    plsc.subcore_barrier()

    b = c * 8 + s // 2
    half = s % 2
    pltpu.sync_copy(ridx_hbm.at[b, pl.ds(half * 2, 2)], idx2)
    pltpu.sync_copy(proj_hbm.at[b, pl.ds(half * 256, 256)], val_v)

    cps = [pltpu.async_copy(val_v.at[pl.ds(k * 128, 128)],
                            scat_hbm.at[idx2.at[k]], sem)
           for k in range(2)]
    for cp in cps:
        cp.wait()


def _kc_body(scat_ref, sp_ref, out_ref):
    t = jnp.swapaxes(scat_ref[0], 0, 1)
    for j in range(4):
        out_ref[j, :C, :] = sp_ref[j]
        out_ref[j, C:, :] = t[j * 32:(j + 1) * 32, :]


def kernel(spatial_info, entity_embeddings, locations, W_proj, b_proj):
    lh = jnp.clip(locations[..., 0], 0, H - 1)
    lw = jnp.clip(locations[..., 1], 0, W - 1)
    fidx = (lh * W + lw).astype(jnp.int32).reshape(B, 1, N)
    sp = spatial_info.reshape(B, C, HW)
    b2 = b_proj.reshape(1, D)

    proj, ridx = pl.pallas_call(
        _ka_body,
        grid=(B,),
        in_specs=[
            pl.BlockSpec((1, N, DIN), lambda b: (b, 0, 0)),
            pl.BlockSpec((1, 1, N), lambda b: (b, 0, 0)),
            pl.BlockSpec((DIN, D), lambda b: (0, 0)),
            pl.BlockSpec((1, D), lambda b: (0, 0)),
        ],
        out_specs=[
            pl.BlockSpec((1, N, D), lambda b: (b, 0, 0)),
            pl.BlockSpec((1, N, 1), lambda b: (b, 0, 0)),
        ],
        out_shape=[
            jax.ShapeDtypeStruct((B, N, D), jnp.float32),
            jax.ShapeDtypeStruct((B, N, 1), jnp.int32),
        ],
    )(entity_embeddings, fidx, W_proj, b2)
    ridx = ridx.reshape(B, 4, 128)

    mesh = plsc.VectorSubcoreMesh(core_axis_name="c", subcore_axis_name="s")
    scat = pl.kernel(
        _ksc_body,
        out_type=jax.ShapeDtypeStruct((SC_ROWS, 32), jnp.float32),
        mesh=mesh,
        scratch_types=[
            pltpu.VMEM((ZBUF, 32), jnp.float32),
            pltpu.VMEM((2, 128), jnp.int32),
            pltpu.VMEM((256, D), jnp.float32),
            pltpu.SemaphoreType.DMA,
        ],
        compiler_params=pltpu.CompilerParams(use_tc_tiling_on_sc=False),
    )(proj, ridx)
    scatq = scat.reshape(NQ, QROWS_PAD, 128)

    out_flat = pl.pallas_call(
        _kc_body,
        grid=(NQ, HW // HWT),
        in_specs=[
            pl.BlockSpec((1, HWT, 128), lambda q, h: (q, h, 0)),
            pl.BlockSpec((4, C, HWT), lambda q, h: (q, 0, h)),
        ],
        out_specs=pl.BlockSpec((4, C + D, HWT), lambda q, h: (q, 0, h)),
        out_shape=jax.ShapeDtypeStruct((B, C + D, HW), jnp.float32),
    )(scatq, sp)
    return out_flat.reshape(B, C + D, H, W)


# speedup vs baseline: 5.2935x; 1.6267x over previous
import jax
import jax.numpy as jnp
from jax import lax
from jax.experimental import pallas as pl
from jax.experimental.pallas import tpu as pltpu
from jax.experimental.pallas import tpu_sc as plsc

B, C, H, W = 16, 20, 128, 128
N, DIN, D = 512, 256, 32
HW = H * W
NQ = B // 4
HWT = 4096
QROWS_PAD = HW + 64
SC_ROWS = NQ * QROWS_PAD * 4
QBASE = QROWS_PAD * 4
NWORK = 32
ZROWS = SC_ROWS // NWORK
ZBUF = 1024


def _ka_body(emb_ref, fidx_ref, w_ref, b_ref, proj_ref, ridx_ref):
    b = pl.program_id(0)
    q = b // 4
    j = b % 4
    emb = emb_ref[0]
    proj = lax.dot_general(
        emb, w_ref[...], (((1,), (0,)), ((), ())),
        preferred_element_type=jnp.float32,
        precision=lax.Precision.HIGHEST,
    ) + b_ref[...]
    proj_ref[0] = jnp.maximum(proj, 0.0)

    row = fidx_ref[0]
    col = jnp.swapaxes(row, 0, 1)
    eq = col == row
    later = (lax.broadcasted_iota(jnp.int32, (N, N), 1)
             > lax.broadcasted_iota(jnp.int32, (N, N), 0))
    dup = jnp.any(eq & later, axis=1, keepdims=True)
    dest = q * QBASE + 4 * col + j
    dump = q * QBASE + 4 * HW + j
    ridx_ref[0] = jnp.where(dup, dump, dest)


def _ksc_body(proj_hbm, ridx_hbm, scat_hbm, zbuf, idx2, val_v, sem):
    c = lax.axis_index("c")
    s = lax.axis_index("s")
    w = c * 16 + s

    z = jnp.zeros((16,), jnp.float32)

    def zstep(i, carry):
        for r in range(8):
            for g in range(2):
                zbuf[i * 8 + r, pl.ds(g * 16, 16)] = z
        return carry

    lax.fori_loop(0, ZBUF // 8, zstep, 0)
    zbase = w * ZROWS
    cps = [pltpu.async_copy(zbuf, scat_hbm.at[pl.ds(zbase + k * ZBUF, ZBUF)], sem)
           for k in range(8)]
    rem = ZROWS - 8 * ZBUF
    cps.append(pltpu.async_copy(zbuf.at[pl.ds(0, rem)],
                                scat_hbm.at[pl.ds(zbase + 8 * ZBUF, rem)], sem))
    for cp in cps:
        cp.wait()

    plsc.subcore_barrier()

    b = c * 8 + s // 2
    half = s % 2
    pltpu.sync_copy(ridx_hbm.at[b, pl.ds(half * 2, 2)], idx2)
    pltpu.sync_copy(proj_hbm.at[b, pl.ds(half * 256, 256)], val_v)

    cps = [pltpu.async_copy(val_v.at[pl.ds(k * 128, 128)],
                            scat_hbm.at[idx2.at[k]], sem)
           for k in range(2)]
    for cp in cps:
        cp.wait()


def _kc_body(scat_ref, sp_ref, out_ref):
    t = jnp.swapaxes(scat_ref[0], 0, 1)
    res = t.reshape(128, HWT // W, W)
    for j in range(4):
        out_ref[j, :C] = sp_ref[j]
        out_ref[j, C:] = res[j * 32:(j + 1) * 32]


def kernel(spatial_info, entity_embeddings, locations, W_proj, b_proj):
    lh = jnp.clip(locations[..., 0], 0, H - 1)
    lw = jnp.clip(locations[..., 1], 0, W - 1)
    fidx = (lh * W + lw).astype(jnp.int32).reshape(B, 1, N)
    b2 = b_proj.reshape(1, D)

    proj, ridx = pl.pallas_call(
        _ka_body,
        grid=(B,),
        in_specs=[
            pl.BlockSpec((1, N, DIN), lambda b: (b, 0, 0)),
            pl.BlockSpec((1, 1, N), lambda b: (b, 0, 0)),
            pl.BlockSpec((DIN, D), lambda b: (0, 0)),
            pl.BlockSpec((1, D), lambda b: (0, 0)),
        ],
        out_specs=[
            pl.BlockSpec((1, N, D), lambda b: (b, 0, 0)),
            pl.BlockSpec((1, N, 1), lambda b: (b, 0, 0)),
        ],
        out_shape=[
            jax.ShapeDtypeStruct((B, N, D), jnp.float32),
            jax.ShapeDtypeStruct((B, N, 1), jnp.int32),
        ],
    )(entity_embeddings, fidx, W_proj, b2)
    ridx = ridx.reshape(B, 4, 128)

    mesh = plsc.VectorSubcoreMesh(core_axis_name="c", subcore_axis_name="s")
    scat = pl.kernel(
        _ksc_body,
        out_type=jax.ShapeDtypeStruct((SC_ROWS, 32), jnp.float32),
        mesh=mesh,
        scratch_types=[
            pltpu.VMEM((ZBUF, 32), jnp.float32),
            pltpu.VMEM((2, 128), jnp.int32),
            pltpu.VMEM((256, D), jnp.float32),
            pltpu.SemaphoreType.DMA,
        ],
        compiler_params=pltpu.CompilerParams(use_tc_tiling_on_sc=False),
    )(proj, ridx)
    scatq = scat.reshape(NQ, QROWS_PAD, 128)

    ht = HWT // W
    out = pl.pallas_call(
        _kc_body,
        grid=(NQ, HW // HWT),
        in_specs=[
            pl.BlockSpec((1, HWT, 128), lambda q, h: (q, h, 0)),
            pl.BlockSpec((4, C, ht, W), lambda q, h: (q, 0, h, 0)),
        ],
        out_specs=pl.BlockSpec((4, C + D, ht, W), lambda q, h: (q, 0, h, 0)),
        out_shape=jax.ShapeDtypeStruct((B, C + D, H, W), jnp.float32),
    )(scatq, spatial_info)
    return out


# speedup vs baseline: 5.6854x; 1.0740x over previous
import jax
import jax.numpy as jnp
from jax import lax
from jax.experimental import pallas as pl
from jax.experimental.pallas import tpu as pltpu
from jax.experimental.pallas import tpu_sc as plsc

B, C, H, W = 16, 20, 128, 128
N, DIN, D = 512, 256, 32
HW = H * W
NQ = B // 4
HWT = 8192
QROWS_PAD = HW + 64
SC_ROWS = NQ * QROWS_PAD * 4
QBASE = QROWS_PAD * 4
NWORK = 32
ZROWS = SC_ROWS // NWORK
ZBUF = 1024


def _ka_body(emb_ref, fidx_ref, w_ref, b_ref, proj_ref, ridx_ref):
    b = pl.program_id(0)
    q = b // 4
    j = b % 4
    emb = emb_ref[0]
    proj = lax.dot_general(
        emb, w_ref[...], (((1,), (0,)), ((), ())),
        preferred_element_type=jnp.float32,
        precision=lax.Precision.HIGHEST,
    ) + b_ref[...]
    proj_ref[0] = jnp.maximum(proj, 0.0)

    row = fidx_ref[0]
    col = jnp.swapaxes(row, 0, 1)
    eq = col == row
    later = (lax.broadcasted_iota(jnp.int32, (N, N), 1)
             > lax.broadcasted_iota(jnp.int32, (N, N), 0))
    dup = jnp.any(eq & later, axis=1, keepdims=True)
    dest = q * QBASE + 4 * col + j
    dump = q * QBASE + 4 * HW + j
    ridx_ref[0] = jnp.where(dup, dump, dest).reshape(4, 128)


def _ksc_body(proj_hbm, ridx_hbm, scat_hbm, zbuf, idx2, val_v, sem):
    c = lax.axis_index("c")
    s = lax.axis_index("s")
    w = c * 16 + s

    z = jnp.zeros((16,), jnp.float32)

    def zstep(i, carry):
        for r in range(8):
            for g in range(2):
                zbuf[i * 8 + r, pl.ds(g * 16, 16)] = z
        return carry

    lax.fori_loop(0, ZBUF // 8, zstep, 0)
    zbase = w * ZROWS
    cps = [pltpu.async_copy(zbuf, scat_hbm.at[pl.ds(zbase + k * ZBUF, ZBUF)], sem)
           for k in range(8)]
    rem = ZROWS - 8 * ZBUF
    cps.append(pltpu.async_copy(zbuf.at[pl.ds(0, rem)],
                                scat_hbm.at[pl.ds(zbase + 8 * ZBUF, rem)], sem))
    for cp in cps:
        cp.wait()

    plsc.subcore_barrier()

    b = c * 8 + s // 2
    half = s % 2
    pltpu.sync_copy(ridx_hbm.at[b, pl.ds(half * 2, 2)], idx2)
    pltpu.sync_copy(proj_hbm.at[b, pl.ds(half * 256, 256)], val_v)

    cps = [pltpu.async_copy(val_v.at[pl.ds(k * 128, 128)],
                            scat_hbm.at[idx2.at[k]], sem)
           for k in range(2)]
    for cp in cps:
        cp.wait()


def _kc_body(scat_ref, sp_ref, out_ref):
    t = jnp.swapaxes(scat_ref[0], 0, 1)
    res = t.reshape(128, HWT // W, W)
    for j in range(4):
        out_ref[j, :C] = sp_ref[j]
        out_ref[j, C:] = res[j * 32:(j + 1) * 32]


def kernel(spatial_info, entity_embeddings, locations, W_proj, b_proj):
    lh = jnp.clip(locations[..., 0], 0, H - 1)
    lw = jnp.clip(locations[..., 1], 0, W - 1)
    fidx = (lh * W + lw).astype(jnp.int32).reshape(B, 1, N)
    b2 = b_proj.reshape(1, D)

    proj, ridx = pl.pallas_call(
        _ka_body,
        grid=(B,),
        in_specs=[
            pl.BlockSpec((1, N, DIN), lambda b: (b, 0, 0)),
            pl.BlockSpec((1, 1, N), lambda b: (b, 0, 0)),
            pl.BlockSpec((DIN, D), lambda b: (0, 0)),
            pl.BlockSpec((1, D), lambda b: (0, 0)),
        ],
        out_specs=[
            pl.BlockSpec((1, N, D), lambda b: (b, 0, 0)),
            pl.BlockSpec((1, 4, 128), lambda b: (b, 0, 0)),
        ],
        out_shape=[
            jax.ShapeDtypeStruct((B, N, D), jnp.float32),
            jax.ShapeDtypeStruct((B, 4, 128), jnp.int32),
        ],
    )(entity_embeddings, fidx, W_proj, b2)

    mesh = plsc.VectorSubcoreMesh(core_axis_name="c", subcore_axis_name="s")
    scat = pl.kernel(
        _ksc_body,
        out_type=jax.ShapeDtypeStruct((SC_ROWS, 32), jnp.float32),
        mesh=mesh,
        scratch_types=[
            pltpu.VMEM((ZBUF, 32), jnp.float32),
            pltpu.VMEM((2, 128), jnp.int32),
            pltpu.VMEM((256, D), jnp.float32),
            pltpu.SemaphoreType.DMA,
        ],
        compiler_params=pltpu.CompilerParams(use_tc_tiling_on_sc=False),
    )(proj, ridx)
    scatq = scat.reshape(NQ, QROWS_PAD, 128)

    ht = HWT // W
    out = pl.pallas_call(
        _kc_body,
        grid=(NQ, HW // HWT),
        in_specs=[
            pl.BlockSpec((1, HWT, 128), lambda q, h: (q, h, 0)),
            pl.BlockSpec((4, C, ht, W), lambda q, h: (q, 0, h, 0)),
        ],
        out_specs=pl.BlockSpec((4, C + D, ht, W), lambda q, h: (q, 0, h, 0)),
        out_shape=jax.ShapeDtypeStruct((B, C + D, H, W), jnp.float32),
    )(scatq, spatial_info)
    return out
